# baseline (speedup 1.0000x reference)
import functools

import jax
import jax.numpy as jnp
from jax.experimental import pallas as pl


def _fps_jax(xyz, npoint):
    xyz = jax.lax.stop_gradient(xyz)
    b, n, _ = xyz.shape

    def step(carry, _):
        dists, farthest = carry
        centroid = jnp.take_along_axis(xyz, farthest[:, None, None], axis=1)
        d = jnp.sum((xyz - centroid) ** 2, axis=-1)
        dists = jnp.minimum(dists, d)
        nxt = jnp.argmax(dists, axis=-1).astype(jnp.int32)
        return (dists, nxt), farthest

    init = (jnp.full((b, n), 1e10, jnp.float32), jnp.zeros((b,), jnp.int32))
    _, idx = jax.lax.scan(step, init, None, length=npoint)
    return jnp.transpose(idx, (1, 0))


def _ball_query_jax(xyz, new_xyz, radius, nsample):
    d2 = jnp.sum((new_xyz[:, :, None, :] - xyz[:, None, :, :]) ** 2, axis=-1)
    n = xyz.shape[1]
    cand = jnp.where(d2 < radius * radius,
                     jnp.arange(n, dtype=jnp.int32)[None, None, :], jnp.int32(n))
    neg_top, _ = jax.lax.top_k(-cand, nsample)
    idx = -neg_top
    first = idx[:, :, :1]
    idx = jnp.where(idx == n, first, idx)
    idx = jnp.where(idx == n, 0, idx)
    return idx


def _gather(points, idx):
    return jax.vmap(lambda p, i: p[i])(points, idx)


def _mlp_max_kernel(g_ref, w1_ref, b1_ref, w2_ref, b2_ref, w3_ref, b3_ref,
                    out_ref, *, pc, s):
    cin = g_ref.shape[-1]
    x = g_ref[0].reshape(pc * s, cin)
    x = jnp.maximum(
        jnp.dot(x, w1_ref[...], preferred_element_type=jnp.float32)
        + b1_ref[0], 0.0)
    x = jnp.maximum(
        jnp.dot(x, w2_ref[...], preferred_element_type=jnp.float32)
        + b2_ref[0], 0.0)
    x = jnp.maximum(
        jnp.dot(x, w3_ref[...], preferred_element_type=jnp.float32)
        + b3_ref[0], 0.0)
    h3 = x.shape[-1]
    out_ref[0] = x.reshape(pc, s, h3).max(axis=1)


def _mlp_max(g, params):
    b, p, s, cin = g.shape
    (w1, b1), (w2, b2), (w3, b3) = params
    h3 = w3.shape[1]
    pc = max(1, min(p, 2048 // s))
    while p % pc:
        pc -= 1
    grid = (b, p // pc)
    b1r, b2r, b3r = (x.reshape(1, -1) for x in (b1, b2, b3))

    def wspec(w):
        return pl.BlockSpec(w.shape, lambda i, j: (0,) * w.ndim)

    return pl.pallas_call(
        functools.partial(_mlp_max_kernel, pc=pc, s=s),
        grid=grid,
        in_specs=[
            pl.BlockSpec((1, pc, s, cin), lambda i, j: (i, j, 0, 0)),
            wspec(w1), wspec(b1r), wspec(w2), wspec(b2r), wspec(w3), wspec(b3r),
        ],
        out_specs=pl.BlockSpec((1, pc, h3), lambda i, j: (i, j, 0)),
        out_shape=jax.ShapeDtypeStruct((b, p, h3), jnp.float32),
    )(g, w1, b1r, w2, b2r, w3, b3r)


def _sa_msg(xyz, features, npoint, radii, nsamples, params_list):
    idx_f = _fps_jax(xyz, npoint)
    new_xyz = _gather(xyz, idx_f)
    outs = []
    for radius, nsample, params in zip(radii, nsamples, params_list):
        gi = _ball_query_jax(xyz, new_xyz, radius, nsample)
        g_xyz = _gather(xyz, gi) - new_xyz[:, :, None, :]
        g_feat = _gather(features, gi)
        g = jnp.concatenate([g_xyz, g_feat], axis=-1)
        outs.append(_mlp_max(g, params))
    return new_xyz, jnp.concatenate(outs, axis=-1)


def kernel(xyz, sa1_params, sa2_params, sa3_params):
    l1_xyz, l1_f = _sa_msg(xyz, xyz, 512, [0.1, 0.2, 0.4], [32, 64, 128],
                           sa1_params)
    l2_xyz, l2_f = _sa_msg(l1_xyz, l1_f, 128, [0.4, 0.8], [64, 128],
                           sa2_params)
    g3 = jnp.concatenate([l2_xyz, l2_f], axis=-1)[:, None, :, :]
    l3 = _mlp_max(g3, sa3_params)
    return jnp.transpose(l3, (0, 2, 1))


# speedup vs baseline: 2.9875x; 2.9875x over previous
import functools

import numpy as np
import jax
import jax.numpy as jnp
from jax.experimental import pallas as pl


def _fps_kernel(xyz_ref, out_ref, *, npoint, rows, cols):
    x = xyz_ref[0, 0]
    y = xyz_ref[0, 1]
    z = xyz_ref[0, 2]
    ri = jax.lax.broadcasted_iota(jnp.int32, (rows, cols), 0)
    ci = jax.lax.broadcasted_iota(jnp.int32, (rows, cols), 1)
    jf = ri * cols + ci

    def step(i, carry):
        dists, far = carry
        out_ref[0, pl.ds(i, 1), :] = jnp.full((1, 1), far, jnp.int32)
        onehot = (jf == far).astype(jnp.float32)
        cx = jnp.sum(x * onehot)
        cy = jnp.sum(y * onehot)
        cz = jnp.sum(z * onehot)
        dx = x - cx
        dy = y - cy
        dz = z - cz
        d = dx * dx + dy * dy + dz * dz
        dists = jnp.minimum(dists, d)
        mx = jnp.max(dists)
        cand = jnp.where(dists == mx, jf, jnp.int32(2 ** 30))
        return dists, jnp.min(cand)

    init = (jnp.full((rows, cols), 1e10, jnp.float32), jnp.int32(0))
    jax.lax.fori_loop(0, npoint, step, init)


def _fps(xyz, npoint):
    b, n, _ = xyz.shape
    rows = 8
    cols = n // rows
    xyz_r = jnp.transpose(xyz, (0, 2, 1)).reshape(b, 3, rows, cols)
    out = pl.pallas_call(
        functools.partial(_fps_kernel, npoint=npoint, rows=rows, cols=cols),
        grid=(b,),
        in_specs=[pl.BlockSpec((1, 3, rows, cols), lambda i: (i, 0, 0, 0))],
        out_specs=pl.BlockSpec((1, npoint, 1), lambda i: (i, 0, 0)),
        out_shape=jax.ShapeDtypeStruct((b, npoint, 1), jnp.int32),
    )(xyz_r)
    return out[:, :, 0]


def _bq_kernel(xyz_ref, cen_ref, tl_ref, tr_ref, *out_refs, branches, pc,
               rows):
    x = xyz_ref[0, 0]
    y = xyz_ref[0, 1]
    z = xyz_ref[0, 2]
    tl = tl_ref[...]
    tr = tr_ref[...]
    lane_col = jax.lax.broadcasted_iota(jnp.int32, (128, 1), 0).astype(jnp.float32)
    row_col = jax.lax.broadcasted_iota(jnp.int32, (rows, 1), 0).astype(jnp.float32)

    def one_center(i, _):
        cx = cen_ref[0, 0, pl.ds(i, 1), :]
        cy = cen_ref[0, 1, pl.ds(i, 1), :]
        cz = cen_ref[0, 2, pl.ds(i, 1), :]
        dx = x - cx[0, 0]
        dy = y - cy[0, 0]
        dz = z - cz[0, 0]
        d2 = dx * dx + dy * dy + dz * dz

        for (r2, ns), out_ref in zip(branches, out_refs):
            mask = (d2 < r2).astype(jnp.float32)
            cs = jnp.dot(mask, tl, preferred_element_type=jnp.float32)
            rc = cs[:, 127:128] + mask[:, 127:128]
            row_off = jnp.dot(tr, rc, preferred_element_type=jnp.float32)
            total = row_off[rows - 1, 0] + rc[rows - 1, 0]
            kio = jax.lax.broadcasted_iota(jnp.int32, (rows, ns), 1).astype(jnp.float32)
            c1 = (row_off <= kio).astype(jnp.float32)
            rk = jnp.sum(c1, axis=0, keepdims=True) - 1.0
            rio = jax.lax.broadcasted_iota(jnp.int32, (rows, ns), 0).astype(jnp.float32)
            rowsel = (rio == rk).astype(jnp.float32)
            rowsel_t = rowsel.T
            ro_k = jnp.dot(rowsel_t, row_off,
                           preferred_element_type=jnp.float32)
            r_col = jnp.dot(rowsel_t, row_col,
                            preferred_element_type=jnp.float32)
            q = jnp.dot(rowsel_t, cs,
                        preferred_element_type=jnp.float32)
            qm = jnp.dot(rowsel_t, mask,
                         preferred_element_type=jnp.float32)
            k_col = jax.lax.broadcasted_iota(jnp.int32, (ns, 1), 0).astype(jnp.float32)
            m_col = k_col - ro_k
            hot = jnp.logical_and(q == m_col, qm > 0.5).astype(jnp.float32)
            l_col = jnp.dot(hot, lane_col,
                            preferred_element_type=jnp.float32)
            idx_col = r_col * 128.0 + l_col
            idx_col = jnp.where(k_col < total, idx_col, idx_col[0, 0])
            out_ref[0, pl.ds(i, 1), :] = idx_col.astype(jnp.int32).T
        return 0

    jax.lax.fori_loop(0, pc, one_center, 0)


def _ball_query(xyz, new_xyz, radii, nsamples):
    b, n, _ = xyz.shape
    p = new_xyz.shape[1]
    rows = n // 128
    pc = 8
    branches = tuple((np.float32(r * r), ns) for r, ns in zip(radii, nsamples))
    xyz_r = jnp.transpose(xyz, (0, 2, 1)).reshape(b, 3, rows, 128)
    cen_r = jnp.transpose(new_xyz, (0, 2, 1)).reshape(b, 3, p, 1)
    tl = jnp.asarray(np.triu(np.ones((128, 128), np.float32), 1))
    tr = jnp.asarray(np.tril(np.ones((rows, rows), np.float32), -1))
    outs = pl.pallas_call(
        functools.partial(_bq_kernel, branches=branches, pc=pc, rows=rows),
        grid=(b, p // pc),
        in_specs=[
            pl.BlockSpec((1, 3, rows, 128), lambda i, j: (i, 0, 0, 0)),
            pl.BlockSpec((1, 3, pc, 1), lambda i, j: (i, 0, j, 0)),
            pl.BlockSpec((128, 128), lambda i, j: (0, 0)),
            pl.BlockSpec((rows, rows), lambda i, j: (0, 0)),
        ],
        out_specs=[
            pl.BlockSpec((1, pc, ns), lambda i, j: (i, j, 0))
            for _, ns in branches
        ],
        out_shape=[
            jax.ShapeDtypeStruct((b, p, ns), jnp.int32) for _, ns in branches
        ],
    )(xyz_r, cen_r, tl, tr)
    return outs


def _gather(points, idx):
    return jax.vmap(lambda pts, i: pts[i])(points, idx)


def _mlp_max_kernel(g_ref, w1_ref, b1_ref, w2_ref, b2_ref, w3_ref, b3_ref,
                    out_ref, *, pc, s):
    cin = g_ref.shape[-1]
    x = g_ref[0].reshape(pc * s, cin)
    x = jnp.maximum(
        jnp.dot(x, w1_ref[...], preferred_element_type=jnp.float32)
        + b1_ref[0], 0.0)
    x = jnp.maximum(
        jnp.dot(x, w2_ref[...], preferred_element_type=jnp.float32)
        + b2_ref[0], 0.0)
    x = jnp.maximum(
        jnp.dot(x, w3_ref[...], preferred_element_type=jnp.float32)
        + b3_ref[0], 0.0)
    h3 = x.shape[-1]
    out_ref[0] = x.reshape(pc, s, h3).max(axis=1)


def _mlp_max(g, params):
    b, p, s, cin = g.shape
    (w1, b1), (w2, b2), (w3, b3) = params
    h3 = w3.shape[1]
    pc = max(1, min(p, 2048 // s))
    while p % pc:
        pc -= 1
    grid = (b, p // pc)
    b1r, b2r, b3r = (v.reshape(1, -1) for v in (b1, b2, b3))

    def wspec(w):
        return pl.BlockSpec(w.shape, lambda i, j: (0,) * w.ndim)

    return pl.pallas_call(
        functools.partial(_mlp_max_kernel, pc=pc, s=s),
        grid=grid,
        in_specs=[
            pl.BlockSpec((1, pc, s, cin), lambda i, j: (i, j, 0, 0)),
            wspec(w1), wspec(b1r), wspec(w2), wspec(b2r), wspec(w3), wspec(b3r),
        ],
        out_specs=pl.BlockSpec((1, pc, h3), lambda i, j: (i, j, 0)),
        out_shape=jax.ShapeDtypeStruct((b, p, h3), jnp.float32),
    )(g, w1, b1r, w2, b2r, w3, b3r)


def _sa_msg(xyz, features, npoint, radii, nsamples, params_list):
    idx_f = _fps(xyz, npoint)
    new_xyz = _gather(xyz, idx_f)
    gidx = _ball_query(xyz, new_xyz, radii, nsamples)
    outs = []
    for gi, params in zip(gidx, params_list):
        g_xyz = _gather(xyz, gi) - new_xyz[:, :, None, :]
        g_feat = _gather(features, gi)
        g = jnp.concatenate([g_xyz, g_feat], axis=-1)
        outs.append(_mlp_max(g, params))
    return new_xyz, jnp.concatenate(outs, axis=-1)


def kernel(xyz, sa1_params, sa2_params, sa3_params):
    l1_xyz, l1_f = _sa_msg(xyz, xyz, 512, [0.1, 0.2, 0.4], [32, 64, 128],
                           sa1_params)
    l2_xyz, l2_f = _sa_msg(l1_xyz, l1_f, 128, [0.4, 0.8], [64, 128],
                           sa2_params)
    g3 = jnp.concatenate([l2_xyz, l2_f], axis=-1)[:, None, :, :]
    l3 = _mlp_max(g3, sa3_params)
    return jnp.transpose(l3, (0, 2, 1))


# speedup vs baseline: 6.2038x; 2.0766x over previous
import functools

import numpy as np
import jax
import jax.numpy as jnp
from jax.experimental import pallas as pl


def _fps_kernel(xyz_ref, out_ref, *, npoint, rows, cols):
    x = xyz_ref[0, 0]
    y = xyz_ref[0, 1]
    z = xyz_ref[0, 2]
    ri = jax.lax.broadcasted_iota(jnp.int32, (rows, cols), 0)
    ci = jax.lax.broadcasted_iota(jnp.int32, (rows, cols), 1)
    jf = ri * cols + ci

    def step(i, carry):
        dists, far = carry
        out_ref[0, pl.ds(i, 1), :] = jnp.full((1, 1), far, jnp.int32)
        onehot = (jf == far).astype(jnp.float32)
        cx = jnp.sum(x * onehot)
        cy = jnp.sum(y * onehot)
        cz = jnp.sum(z * onehot)
        dx = x - cx
        dy = y - cy
        dz = z - cz
        d = dx * dx + dy * dy + dz * dz
        dists = jnp.minimum(dists, d)
        mx = jnp.max(dists)
        cand = jnp.where(dists == mx, jf, jnp.int32(2 ** 30))
        return dists, jnp.min(cand)

    init = (jnp.full((rows, cols), 1e10, jnp.float32), jnp.int32(0))
    jax.lax.fori_loop(0, npoint, step, init)


def _fps(xyz, npoint):
    b, n, _ = xyz.shape
    rows = 8
    cols = n // rows
    xyz_r = jnp.transpose(xyz, (0, 2, 1)).reshape(b, 3, rows, cols)
    out = pl.pallas_call(
        functools.partial(_fps_kernel, npoint=npoint, rows=rows, cols=cols),
        grid=(b,),
        in_specs=[pl.BlockSpec((1, 3, rows, cols), lambda i: (i, 0, 0, 0))],
        out_specs=pl.BlockSpec((1, npoint, 1), lambda i: (i, 0, 0)),
        out_shape=jax.ShapeDtypeStruct((b, npoint, 1), jnp.int32),
    )(xyz_r)
    return out[:, :, 0]


def _selection(d2, r2, ns, tl, tr, rows):
    mask = (d2 < r2).astype(jnp.float32)
    cs = jnp.dot(mask, tl, preferred_element_type=jnp.float32)
    rc = cs[:, 127:128] + mask[:, 127:128]
    row_off = jnp.dot(tr, rc, preferred_element_type=jnp.float32)
    total = row_off[rows - 1, 0] + rc[rows - 1, 0]
    kio = jax.lax.broadcasted_iota(jnp.int32, (rows, ns), 1).astype(jnp.float32)
    c1 = (row_off <= kio).astype(jnp.float32)
    rk = jnp.sum(c1, axis=0, keepdims=True) - 1.0
    rio = jax.lax.broadcasted_iota(jnp.int32, (rows, ns), 0).astype(jnp.float32)
    rowsel = (rio == rk).astype(jnp.float32)
    rowsel_t = rowsel.T
    ro_k = jnp.dot(rowsel_t, row_off,
                   preferred_element_type=jnp.float32)
    k_col = jax.lax.broadcasted_iota(jnp.int32, (ns, 1), 0).astype(jnp.float32)
    m_col = k_col - ro_k
    return rowsel_t, cs, mask, m_col, k_col, total


def _bg1_kernel(xyz_ref, cen_ref, tl_ref, tr_ref, *out_refs, branches, pc,
                rows):
    x = xyz_ref[0, 0]
    y = xyz_ref[0, 1]
    z = xyz_ref[0, 2]
    tl = tl_ref[...]
    tr = tr_ref[...]

    def one_center(i, _):
        cx = cen_ref[0, 0, pl.ds(i, 1), :][0, 0]
        cy = cen_ref[0, 1, pl.ds(i, 1), :][0, 0]
        cz = cen_ref[0, 2, pl.ds(i, 1), :][0, 0]
        dx = x - cx
        dy = y - cy
        dz = z - cz
        d2 = dx * dx + dy * dy + dz * dz

        for (r2, ns), out_ref in zip(branches, out_refs):
            rowsel_t, cs, mask, m_col, k_col, total = _selection(
                d2, r2, ns, tl, tr, rows)
            big = jnp.concatenate([cs, mask, x, y, z], axis=1)
            gath = jnp.dot(rowsel_t, big, precision=jax.lax.Precision.HIGHEST,
                           preferred_element_type=jnp.float32)
            q = gath[:, 0:128]
            qm = gath[:, 128:256]
            hot = jnp.logical_and(q == m_col, qm > 0.5).astype(jnp.float32)
            gx = jnp.sum(gath[:, 256:384] * hot, axis=1, keepdims=True)
            gy = jnp.sum(gath[:, 384:512] * hot, axis=1, keepdims=True)
            gz = jnp.sum(gath[:, 512:640] * hot, axis=1, keepdims=True)
            valid = k_col < total
            gx = jnp.where(valid, gx, gx[0, 0])
            gy = jnp.where(valid, gy, gy[0, 0])
            gz = jnp.where(valid, gz, gz[0, 0])
            g = jnp.concatenate(
                [gx - cx, gy - cy, gz - cz, gx, gy, gz], axis=1)
            out_ref[0, pl.ds(i, 1)] = g[None]
        return 0

    jax.lax.fori_loop(0, pc, one_center, 0)


def _ball_group1(xyz, new_xyz, radii, nsamples):
    b, n, _ = xyz.shape
    p = new_xyz.shape[1]
    rows = n // 128
    pc = 8
    branches = tuple((np.float32(r * r), ns) for r, ns in zip(radii, nsamples))
    xyz_r = jnp.transpose(xyz, (0, 2, 1)).reshape(b, 3, rows, 128)
    cen_r = jnp.transpose(new_xyz, (0, 2, 1)).reshape(b, 3, p, 1)
    tl = jnp.asarray(np.triu(np.ones((128, 128), np.float32), 1))
    tr = jnp.asarray(np.tril(np.ones((rows, rows), np.float32), -1))
    return pl.pallas_call(
        functools.partial(_bg1_kernel, branches=branches, pc=pc, rows=rows),
        grid=(b, p // pc),
        in_specs=[
            pl.BlockSpec((1, 3, rows, 128), lambda i, j: (i, 0, 0, 0)),
            pl.BlockSpec((1, 3, pc, 1), lambda i, j: (i, 0, j, 0)),
            pl.BlockSpec((128, 128), lambda i, j: (0, 0)),
            pl.BlockSpec((rows, rows), lambda i, j: (0, 0)),
        ],
        out_specs=[
            pl.BlockSpec((1, pc, ns, 6), lambda i, j: (i, j, 0, 0))
            for _, ns in branches
        ],
        out_shape=[
            jax.ShapeDtypeStruct((b, p, ns, 6), jnp.float32)
            for _, ns in branches
        ],
    )(xyz_r, cen_r, tl, tr)


def _bg2_kernel(xyz_ref, cen_ref, tl_ref, tr_ref, tab_ref, *out_refs,
                branches, pc, rows, c):
    x = xyz_ref[0, 0]
    y = xyz_ref[0, 1]
    z = xyz_ref[0, 2]
    tl = tl_ref[...]
    tr = tr_ref[...]
    tab = tab_ref[0]
    n = rows * 128
    lane_col = jax.lax.broadcasted_iota(jnp.int32, (128, 1), 0).astype(jnp.float32)
    row_col = jax.lax.broadcasted_iota(jnp.int32, (rows, 1), 0).astype(jnp.float32)
    cio = jax.lax.broadcasted_iota(jnp.int32, (1, c), 1)

    def one_center(i, _):
        cx = cen_ref[0, 0, pl.ds(i, 1), :][0, 0]
        cy = cen_ref[0, 1, pl.ds(i, 1), :][0, 0]
        cz = cen_ref[0, 2, pl.ds(i, 1), :][0, 0]
        dx = x - cx
        dy = y - cy
        dz = z - cz
        d2 = dx * dx + dy * dy + dz * dz
        cpad = (cx * (cio == 0) + cy * (cio == 1) + cz * (cio == 2)
                ).astype(jnp.float32)

        for (r2, ns), out_ref in zip(branches, out_refs):
            rowsel_t, cs, mask, m_col, k_col, total = _selection(
                d2, r2, ns, tl, tr, rows)
            big = jnp.concatenate([cs, mask], axis=1)
            gath = jnp.dot(rowsel_t, big,
                           preferred_element_type=jnp.float32)
            hot = jnp.logical_and(gath[:, 0:128] == m_col,
                                  gath[:, 128:256] > 0.5).astype(jnp.float32)
            l_col = jnp.dot(hot, lane_col,
                            preferred_element_type=jnp.float32)
            r_col = jnp.dot(rowsel_t, row_col,
                            preferred_element_type=jnp.float32)
            idx_col = r_col * 128.0 + l_col
            idx_col = jnp.where(k_col < total, idx_col, idx_col[0, 0])
            jio = jax.lax.broadcasted_iota(jnp.int32, (ns, n), 1).astype(jnp.float32)
            ph = (jio == idx_col).astype(jnp.float32)
            g = jnp.dot(ph, tab, precision=jax.lax.Precision.HIGHEST,
                        preferred_element_type=jnp.float32) - cpad
            out_ref[0, pl.ds(i, 1)] = g[None]
        return 0

    jax.lax.fori_loop(0, pc, one_center, 0)


def _ball_group2(xyz, new_xyz, table, radii, nsamples):
    b, n, _ = xyz.shape
    p = new_xyz.shape[1]
    c = table.shape[-1]
    rows = n // 128
    pc = 8
    branches = tuple((np.float32(r * r), ns) for r, ns in zip(radii, nsamples))
    xyz_r = jnp.transpose(xyz, (0, 2, 1)).reshape(b, 3, rows, 128)
    cen_r = jnp.transpose(new_xyz, (0, 2, 1)).reshape(b, 3, p, 1)
    tl = jnp.asarray(np.triu(np.ones((128, 128), np.float32), 1))
    tr = jnp.asarray(np.tril(np.ones((rows, rows), np.float32), -1))
    return pl.pallas_call(
        functools.partial(_bg2_kernel, branches=branches, pc=pc, rows=rows,
                          c=c),
        grid=(b, p // pc),
        in_specs=[
            pl.BlockSpec((1, 3, rows, 128), lambda i, j: (i, 0, 0, 0)),
            pl.BlockSpec((1, 3, pc, 1), lambda i, j: (i, 0, j, 0)),
            pl.BlockSpec((128, 128), lambda i, j: (0, 0)),
            pl.BlockSpec((rows, rows), lambda i, j: (0, 0)),
            pl.BlockSpec((1, n, c), lambda i, j: (i, 0, 0)),
        ],
        out_specs=[
            pl.BlockSpec((1, pc, ns, c), lambda i, j: (i, j, 0, 0))
            for _, ns in branches
        ],
        out_shape=[
            jax.ShapeDtypeStruct((b, p, ns, c), jnp.float32)
            for _, ns in branches
        ],
    )(xyz_r, cen_r, tl, tr, table)


def _gather(points, idx):
    return jax.vmap(lambda pts, i: pts[i])(points, idx)


def _mlp_max_kernel(g_ref, w1_ref, b1_ref, w2_ref, b2_ref, w3_ref, b3_ref,
                    out_ref, *, pc, s):
    cin = g_ref.shape[-1]
    x = g_ref[0].reshape(pc * s, cin)
    x = jnp.maximum(
        jnp.dot(x, w1_ref[...], preferred_element_type=jnp.float32)
        + b1_ref[0], 0.0)
    x = jnp.maximum(
        jnp.dot(x, w2_ref[...], preferred_element_type=jnp.float32)
        + b2_ref[0], 0.0)
    x = jnp.maximum(
        jnp.dot(x, w3_ref[...], preferred_element_type=jnp.float32)
        + b3_ref[0], 0.0)
    h3 = x.shape[-1]
    out_ref[0] = x.reshape(pc, s, h3).max(axis=1)


def _mlp_max(g, params):
    b, p, s, cin = g.shape
    (w1, b1), (w2, b2), (w3, b3) = params
    h3 = w3.shape[1]
    pc = max(1, min(p, 2048 // s))
    while p % pc:
        pc -= 1
    grid = (b, p // pc)
    b1r, b2r, b3r = (v.reshape(1, -1) for v in (b1, b2, b3))

    def wspec(w):
        return pl.BlockSpec(w.shape, lambda i, j: (0,) * w.ndim)

    return pl.pallas_call(
        functools.partial(_mlp_max_kernel, pc=pc, s=s),
        grid=grid,
        in_specs=[
            pl.BlockSpec((1, pc, s, cin), lambda i, j: (i, j, 0, 0)),
            wspec(w1), wspec(b1r), wspec(w2), wspec(b2r), wspec(w3), wspec(b3r),
        ],
        out_specs=pl.BlockSpec((1, pc, h3), lambda i, j: (i, j, 0)),
        out_shape=jax.ShapeDtypeStruct((b, p, h3), jnp.float32),
    )(g, w1, b1r, w2, b2r, w3, b3r)


def kernel(xyz, sa1_params, sa2_params, sa3_params):
    idx1 = _fps(xyz, 512)
    l1_xyz = _gather(xyz, idx1)
    g1 = _ball_group1(xyz, l1_xyz, [0.1, 0.2, 0.4], [32, 64, 128])
    l1_f = jnp.concatenate(
        [_mlp_max(g, p) for g, p in zip(g1, sa1_params)], axis=-1)

    idx2 = _fps(l1_xyz, 128)
    l2_xyz = _gather(l1_xyz, idx2)
    table = jnp.concatenate([l1_xyz, l1_f], axis=-1)
    g2 = _ball_group2(l1_xyz, l2_xyz, table, [0.4, 0.8], [64, 128])
    l2_f = jnp.concatenate(
        [_mlp_max(g, p) for g, p in zip(g2, sa2_params)], axis=-1)

    g3 = jnp.concatenate([l2_xyz, l2_f], axis=-1)[:, None, :, :]
    l3 = _mlp_max(g3, sa3_params)
    return jnp.transpose(l3, (0, 2, 1))


# speedup vs baseline: 6.2701x; 1.0107x over previous
import functools

import numpy as np
import jax
import jax.numpy as jnp
from jax.experimental import pallas as pl


def _fps_kernel(xyz_ref, out_ref, *, npoint, rows, cols):
    x = xyz_ref[0, 0]
    y = xyz_ref[0, 1]
    z = xyz_ref[0, 2]
    ri = jax.lax.broadcasted_iota(jnp.int32, (rows, cols), 0)
    ci = jax.lax.broadcasted_iota(jnp.int32, (rows, cols), 1)
    jf = ri * cols + ci

    def step(i, carry):
        dists, far = carry
        out_ref[0, pl.ds(i, 1), :] = jnp.full((1, 1), far, jnp.int32)
        onehot = (jf == far).astype(jnp.float32)
        cx = jnp.sum(x * onehot)
        cy = jnp.sum(y * onehot)
        cz = jnp.sum(z * onehot)
        dx = x - cx
        dy = y - cy
        dz = z - cz
        d = dx * dx + dy * dy + dz * dz
        dists = jnp.minimum(dists, d)
        mx = jnp.max(dists)
        cand = jnp.where(dists == mx, jf, jnp.int32(2 ** 30))
        return dists, jnp.min(cand)

    init = (jnp.full((rows, cols), 1e10, jnp.float32), jnp.int32(0))
    jax.lax.fori_loop(0, npoint, step, init)


def _fps(xyz, npoint):
    b, n, _ = xyz.shape
    rows = 8
    cols = n // rows
    xyz_r = jnp.transpose(xyz, (0, 2, 1)).reshape(b, 3, rows, cols)
    out = pl.pallas_call(
        functools.partial(_fps_kernel, npoint=npoint, rows=rows, cols=cols),
        grid=(b,),
        in_specs=[pl.BlockSpec((1, 3, rows, cols), lambda i: (i, 0, 0, 0))],
        out_specs=pl.BlockSpec((1, npoint, 1), lambda i: (i, 0, 0)),
        out_shape=jax.ShapeDtypeStruct((b, npoint, 1), jnp.int32),
    )(xyz_r)
    return out[:, :, 0]


def _selection(d2, r2, ns, tl, tr, rows):
    mask = (d2 < r2).astype(jnp.bfloat16)
    cs = jnp.dot(mask, tl, preferred_element_type=jnp.float32)
    rc = cs[:, 127:128] + mask[:, 127:128].astype(jnp.float32)
    row_off = jnp.dot(tr.astype(jnp.float32), rc,
                      preferred_element_type=jnp.float32)
    total = row_off[rows - 1, 0] + rc[rows - 1, 0]
    kio = jax.lax.broadcasted_iota(jnp.int32, (rows, ns), 1).astype(jnp.float32)
    c1 = (row_off <= kio).astype(jnp.float32)
    rk = jnp.sum(c1, axis=0, keepdims=True) - 1.0
    rio = jax.lax.broadcasted_iota(jnp.int32, (rows, ns), 0).astype(jnp.float32)
    rowsel = (rio == rk).astype(jnp.bfloat16)
    rowsel_t = rowsel.T
    ro_k = jnp.dot(rowsel_t.astype(jnp.float32), row_off,
                   preferred_element_type=jnp.float32)
    k_col = jax.lax.broadcasted_iota(jnp.int32, (ns, 1), 0).astype(jnp.float32)
    m_col = k_col - ro_k
    return rowsel_t, cs, mask, m_col, k_col, total


def _bg1_kernel(xyz_ref, cen_ref, tl_ref, tr_ref, *out_refs, branches, pc,
                rows):
    x = xyz_ref[0, 0]
    y = xyz_ref[0, 1]
    z = xyz_ref[0, 2]
    tl = tl_ref[...]
    tr = tr_ref[...]
    xyz_cat = jnp.concatenate([x, y, z], axis=1)

    for i in range(pc):
        cx = cen_ref[0, 0, i, 0]
        cy = cen_ref[0, 1, i, 0]
        cz = cen_ref[0, 2, i, 0]
        dx = x - cx
        dy = y - cy
        dz = z - cz
        d2 = dx * dx + dy * dy + dz * dz

        for (r2, ns), out_ref in zip(branches, out_refs):
            rowsel_t, cs, mask, m_col, k_col, total = _selection(
                d2, r2, ns, tl, tr, rows)
            big = jnp.concatenate([cs.astype(jnp.bfloat16), mask], axis=1)
            gath = jnp.dot(rowsel_t, big,
                           preferred_element_type=jnp.float32)
            gxyz = jnp.dot(rowsel_t.astype(jnp.float32), xyz_cat,
                           precision=jax.lax.Precision.HIGHEST,
                           preferred_element_type=jnp.float32)
            q = gath[:, 0:128]
            qm = gath[:, 128:256]
            hot = jnp.logical_and(q == m_col, qm > 0.5).astype(jnp.float32)
            gx = jnp.sum(gxyz[:, 0:128] * hot, axis=1, keepdims=True)
            gy = jnp.sum(gxyz[:, 128:256] * hot, axis=1, keepdims=True)
            gz = jnp.sum(gxyz[:, 256:384] * hot, axis=1, keepdims=True)
            valid = k_col < total
            gx = jnp.where(valid, gx, gx[0, 0])
            gy = jnp.where(valid, gy, gy[0, 0])
            gz = jnp.where(valid, gz, gz[0, 0])
            g = jnp.concatenate(
                [gx - cx, gy - cy, gz - cz, gx, gy, gz], axis=1)
            out_ref[0, i] = g


def _ball_group1(xyz, new_xyz, radii, nsamples):
    b, n, _ = xyz.shape
    p = new_xyz.shape[1]
    rows = n // 128
    pc = 8
    branches = tuple((np.float32(r * r), ns) for r, ns in zip(radii, nsamples))
    xyz_r = jnp.transpose(xyz, (0, 2, 1)).reshape(b, 3, rows, 128)
    cen_r = jnp.transpose(new_xyz, (0, 2, 1)).reshape(b, 3, p, 1)
    tl = jnp.asarray(np.triu(np.ones((128, 128), np.float32), 1), dtype=jnp.bfloat16)
    tr = jnp.asarray(np.tril(np.ones((rows, rows), np.float32), -1))
    return pl.pallas_call(
        functools.partial(_bg1_kernel, branches=branches, pc=pc, rows=rows),
        grid=(b, p // pc),
        in_specs=[
            pl.BlockSpec((1, 3, rows, 128), lambda i, j: (i, 0, 0, 0)),
            pl.BlockSpec((1, 3, pc, 1), lambda i, j: (i, 0, j, 0)),
            pl.BlockSpec((128, 128), lambda i, j: (0, 0)),
            pl.BlockSpec((rows, rows), lambda i, j: (0, 0)),
        ],
        out_specs=[
            pl.BlockSpec((1, pc, ns, 6), lambda i, j: (i, j, 0, 0))
            for _, ns in branches
        ],
        out_shape=[
            jax.ShapeDtypeStruct((b, p, ns, 6), jnp.float32)
            for _, ns in branches
        ],
    )(xyz_r, cen_r, tl, tr)


def _bg2_kernel(xyz_ref, cen_ref, tl_ref, tr_ref, tab_ref, *out_refs,
                branches, pc, rows, c):
    x = xyz_ref[0, 0]
    y = xyz_ref[0, 1]
    z = xyz_ref[0, 2]
    tl = tl_ref[...]
    tr = tr_ref[...]
    tab = tab_ref[0]
    n = rows * 128
    lane_col = jax.lax.broadcasted_iota(jnp.int32, (128, 1), 0).astype(jnp.bfloat16)
    row_col = jax.lax.broadcasted_iota(jnp.int32, (rows, 1), 0).astype(jnp.bfloat16)
    cio = jax.lax.broadcasted_iota(jnp.int32, (1, c), 1)

    for i in range(pc):
        cx = cen_ref[0, 0, i, 0]
        cy = cen_ref[0, 1, i, 0]
        cz = cen_ref[0, 2, i, 0]
        dx = x - cx
        dy = y - cy
        dz = z - cz
        d2 = dx * dx + dy * dy + dz * dz
        cpad = (cx * (cio == 0) + cy * (cio == 1) + cz * (cio == 2)
                ).astype(jnp.float32)

        for (r2, ns), out_ref in zip(branches, out_refs):
            rowsel_t, cs, mask, m_col, k_col, total = _selection(
                d2, r2, ns, tl, tr, rows)
            big = jnp.concatenate([cs.astype(jnp.bfloat16), mask], axis=1)
            gath = jnp.dot(rowsel_t, big,
                           preferred_element_type=jnp.float32)
            hot = jnp.logical_and(gath[:, 0:128] == m_col,
                                  gath[:, 128:256] > 0.5).astype(jnp.bfloat16)
            l_col = jnp.dot(hot, lane_col,
                            preferred_element_type=jnp.float32)
            r_col = jnp.dot(rowsel_t, row_col,
                            preferred_element_type=jnp.float32)
            idx_col = r_col * 128.0 + l_col
            idx_col = jnp.where(k_col < total, idx_col, idx_col[0, 0])
            jio = jax.lax.broadcasted_iota(jnp.int32, (ns, n), 1).astype(jnp.float32)
            ph = (jio == idx_col).astype(jnp.float32)
            g = jnp.dot(ph, tab, precision=jax.lax.Precision.HIGHEST,
                        preferred_element_type=jnp.float32) - cpad
            out_ref[0, i] = g


def _ball_group2(xyz, new_xyz, table, radii, nsamples):
    b, n, _ = xyz.shape
    p = new_xyz.shape[1]
    c = table.shape[-1]
    rows = n // 128
    pc = 8
    branches = tuple((np.float32(r * r), ns) for r, ns in zip(radii, nsamples))
    xyz_r = jnp.transpose(xyz, (0, 2, 1)).reshape(b, 3, rows, 128)
    cen_r = jnp.transpose(new_xyz, (0, 2, 1)).reshape(b, 3, p, 1)
    tl = jnp.asarray(np.triu(np.ones((128, 128), np.float32), 1), dtype=jnp.bfloat16)
    tr = jnp.asarray(np.tril(np.ones((rows, rows), np.float32), -1))
    return pl.pallas_call(
        functools.partial(_bg2_kernel, branches=branches, pc=pc, rows=rows,
                          c=c),
        grid=(b, p // pc),
        in_specs=[
            pl.BlockSpec((1, 3, rows, 128), lambda i, j: (i, 0, 0, 0)),
            pl.BlockSpec((1, 3, pc, 1), lambda i, j: (i, 0, j, 0)),
            pl.BlockSpec((128, 128), lambda i, j: (0, 0)),
            pl.BlockSpec((rows, rows), lambda i, j: (0, 0)),
            pl.BlockSpec((1, n, c), lambda i, j: (i, 0, 0)),
        ],
        out_specs=[
            pl.BlockSpec((1, pc, ns, c), lambda i, j: (i, j, 0, 0))
            for _, ns in branches
        ],
        out_shape=[
            jax.ShapeDtypeStruct((b, p, ns, c), jnp.float32)
            for _, ns in branches
        ],
    )(xyz_r, cen_r, tl, tr, table)


def _gather(points, idx):
    return jax.vmap(lambda pts, i: pts[i])(points, idx)


def _mlp_max_kernel(g_ref, w1_ref, b1_ref, w2_ref, b2_ref, w3_ref, b3_ref,
                    out_ref, *, pc, s):
    cin = g_ref.shape[-1]
    x = g_ref[0].reshape(pc * s, cin)
    x = jnp.maximum(
        jnp.dot(x, w1_ref[...], preferred_element_type=jnp.float32)
        + b1_ref[0], 0.0)
    x = jnp.maximum(
        jnp.dot(x, w2_ref[...], preferred_element_type=jnp.float32)
        + b2_ref[0], 0.0)
    x = jnp.maximum(
        jnp.dot(x, w3_ref[...], preferred_element_type=jnp.float32)
        + b3_ref[0], 0.0)
    h3 = x.shape[-1]
    out_ref[0] = x.reshape(pc, s, h3).max(axis=1)


def _mlp_max(g, params):
    b, p, s, cin = g.shape
    (w1, b1), (w2, b2), (w3, b3) = params
    h3 = w3.shape[1]
    pc = max(1, min(p, 2048 // s))
    while p % pc:
        pc -= 1
    grid = (b, p // pc)
    b1r, b2r, b3r = (v.reshape(1, -1) for v in (b1, b2, b3))

    def wspec(w):
        return pl.BlockSpec(w.shape, lambda i, j: (0,) * w.ndim)

    return pl.pallas_call(
        functools.partial(_mlp_max_kernel, pc=pc, s=s),
        grid=grid,
        in_specs=[
            pl.BlockSpec((1, pc, s, cin), lambda i, j: (i, j, 0, 0)),
            wspec(w1), wspec(b1r), wspec(w2), wspec(b2r), wspec(w3), wspec(b3r),
        ],
        out_specs=pl.BlockSpec((1, pc, h3), lambda i, j: (i, j, 0)),
        out_shape=jax.ShapeDtypeStruct((b, p, h3), jnp.float32),
    )(g, w1, b1r, w2, b2r, w3, b3r)


def kernel(xyz, sa1_params, sa2_params, sa3_params):
    idx1 = _fps(xyz, 512)
    l1_xyz = _gather(xyz, idx1)
    g1 = _ball_group1(xyz, l1_xyz, [0.1, 0.2, 0.4], [32, 64, 128])
    l1_f = jnp.concatenate(
        [_mlp_max(g, p) for g, p in zip(g1, sa1_params)], axis=-1)

    idx2 = _fps(l1_xyz, 128)
    l2_xyz = _gather(l1_xyz, idx2)
    table = jnp.concatenate([l1_xyz, l1_f], axis=-1)
    g2 = _ball_group2(l1_xyz, l2_xyz, table, [0.4, 0.8], [64, 128])
    l2_f = jnp.concatenate(
        [_mlp_max(g, p) for g, p in zip(g2, sa2_params)], axis=-1)

    g3 = jnp.concatenate([l2_xyz, l2_f], axis=-1)[:, None, :, :]
    l3 = _mlp_max(g3, sa3_params)
    return jnp.transpose(l3, (0, 2, 1))


# speedup vs baseline: 6.7359x; 1.0743x over previous
import functools

import numpy as np
import jax
import jax.numpy as jnp
from jax.experimental import pallas as pl


def _fps_kernel(xyz_ref, out_ref, *, npoint, rows, cols):
    x = xyz_ref[0, 0]
    y = xyz_ref[0, 1]
    z = xyz_ref[0, 2]
    ri = jax.lax.broadcasted_iota(jnp.int32, (rows, cols), 0)
    ci = jax.lax.broadcasted_iota(jnp.int32, (rows, cols), 1)
    jf = ri * cols + ci

    def step(i, carry):
        dists, far = carry
        out_ref[0, pl.ds(i, 1), :] = jnp.full((1, 1), far, jnp.int32)
        onehot = (jf == far).astype(jnp.float32)
        cx = jnp.sum(x * onehot)
        cy = jnp.sum(y * onehot)
        cz = jnp.sum(z * onehot)
        dx = x - cx
        dy = y - cy
        dz = z - cz
        d = dx * dx + dy * dy + dz * dz
        dists = jnp.minimum(dists, d)
        mx = jnp.max(dists)
        cand = jnp.where(dists == mx, jf, jnp.int32(2 ** 30))
        return dists, jnp.min(cand)

    init = (jnp.full((rows, cols), 1e10, jnp.float32), jnp.int32(0))
    jax.lax.fori_loop(0, npoint, step, init)


def _fps(xyz, npoint):
    b, n, _ = xyz.shape
    rows = 8
    cols = n // rows
    xyz_r = jnp.transpose(xyz, (0, 2, 1)).reshape(b, 3, rows, cols)
    out = pl.pallas_call(
        functools.partial(_fps_kernel, npoint=npoint, rows=rows, cols=cols),
        grid=(b,),
        in_specs=[pl.BlockSpec((1, 3, rows, cols), lambda i: (i, 0, 0, 0))],
        out_specs=pl.BlockSpec((1, npoint, 1), lambda i: (i, 0, 0)),
        out_shape=jax.ShapeDtypeStruct((b, npoint, 1), jnp.int32),
    )(xyz_r)
    return out[:, :, 0]


def _selection(d2, r2, ns, tl, tr, rows):
    mask = (d2 < r2).astype(jnp.bfloat16)
    cs = jnp.dot(mask, tl, preferred_element_type=jnp.float32)
    rc = cs[:, 127:128] + mask[:, 127:128].astype(jnp.float32)
    row_off = jnp.dot(tr.astype(jnp.float32), rc,
                      preferred_element_type=jnp.float32)
    total = row_off[rows - 1, 0] + rc[rows - 1, 0]
    grank = cs + row_off
    kio = jax.lax.broadcasted_iota(jnp.int32, (rows, ns), 1).astype(jnp.float32)
    c1 = (row_off <= kio).astype(jnp.float32)
    c1s = jnp.concatenate([c1[1:], jnp.zeros((1, ns), jnp.float32)], axis=0)
    rowsel_t = (c1 - c1s).T
    k_col = jax.lax.broadcasted_iota(jnp.int32, (ns, 1), 0).astype(jnp.float32)
    return rowsel_t, grank, mask, k_col, total


def _bg1_kernel(xyz_ref, cen_ref, tl_ref, tr_ref, *out_refs, branches, pc,
                rows):
    x = xyz_ref[0, 0]
    y = xyz_ref[0, 1]
    z = xyz_ref[0, 2]
    tl = tl_ref[...]
    tr = tr_ref[...]
    xyz_cat = jnp.concatenate([x, y, z], axis=1)

    for i in range(pc):
        cx = cen_ref[0, 0, i, 0]
        cy = cen_ref[0, 1, i, 0]
        cz = cen_ref[0, 2, i, 0]
        dx = x - cx
        dy = y - cy
        dz = z - cz
        d2 = dx * dx + dy * dy + dz * dz

        for (r2, ns), out_ref in zip(branches, out_refs):
            rowsel_t, grank, mask, k_col, total = _selection(
                d2, r2, ns, tl, tr, rows)
            big = jnp.concatenate([grank, mask.astype(jnp.float32)], axis=1)
            gath = jnp.dot(rowsel_t, big,
                           preferred_element_type=jnp.float32)
            gxyz = jnp.dot(rowsel_t, xyz_cat,
                           precision=jax.lax.Precision.HIGHEST,
                           preferred_element_type=jnp.float32)
            q = gath[:, 0:128]
            qm = gath[:, 128:256]
            hot = jnp.logical_and(q == k_col, qm > 0.5).astype(jnp.float32)
            gx = jnp.sum(gxyz[:, 0:128] * hot, axis=1, keepdims=True)
            gy = jnp.sum(gxyz[:, 128:256] * hot, axis=1, keepdims=True)
            gz = jnp.sum(gxyz[:, 256:384] * hot, axis=1, keepdims=True)
            valid = k_col < total
            gx = jnp.where(valid, gx, gx[0, 0])
            gy = jnp.where(valid, gy, gy[0, 0])
            gz = jnp.where(valid, gz, gz[0, 0])
            g = jnp.concatenate(
                [gx - cx, gy - cy, gz - cz, gx, gy, gz], axis=1)
            out_ref[0, i] = g


def _ball_group1(xyz, new_xyz, radii, nsamples):
    b, n, _ = xyz.shape
    p = new_xyz.shape[1]
    rows = n // 128
    pc = 8
    branches = tuple((np.float32(r * r), ns) for r, ns in zip(radii, nsamples))
    xyz_r = jnp.transpose(xyz, (0, 2, 1)).reshape(b, 3, rows, 128)
    cen_r = jnp.transpose(new_xyz, (0, 2, 1)).reshape(b, 3, p, 1)
    tl = jnp.asarray(np.triu(np.ones((128, 128), np.float32), 1), dtype=jnp.bfloat16)
    tr = jnp.asarray(np.tril(np.ones((rows, rows), np.float32), -1))
    return pl.pallas_call(
        functools.partial(_bg1_kernel, branches=branches, pc=pc, rows=rows),
        grid=(b, p // pc),
        in_specs=[
            pl.BlockSpec((1, 3, rows, 128), lambda i, j: (i, 0, 0, 0)),
            pl.BlockSpec((1, 3, pc, 1), lambda i, j: (i, 0, j, 0)),
            pl.BlockSpec((128, 128), lambda i, j: (0, 0)),
            pl.BlockSpec((rows, rows), lambda i, j: (0, 0)),
        ],
        out_specs=[
            pl.BlockSpec((1, pc, ns, 6), lambda i, j: (i, j, 0, 0))
            for _, ns in branches
        ],
        out_shape=[
            jax.ShapeDtypeStruct((b, p, ns, 6), jnp.float32)
            for _, ns in branches
        ],
    )(xyz_r, cen_r, tl, tr)


def _bg2_kernel(xyz_ref, cen_ref, tl_ref, tr_ref, tab_ref, *out_refs,
                branches, pc, rows, c):
    x = xyz_ref[0, 0]
    y = xyz_ref[0, 1]
    z = xyz_ref[0, 2]
    tl = tl_ref[...]
    tr = tr_ref[...]
    tab = tab_ref[0]
    n = rows * 128
    lane_col = jax.lax.broadcasted_iota(jnp.int32, (128, 1), 0).astype(jnp.float32)
    row_col = jax.lax.broadcasted_iota(jnp.int32, (rows, 1), 0).astype(jnp.float32)
    cio = jax.lax.broadcasted_iota(jnp.int32, (1, c), 1)

    for i in range(pc):
        cx = cen_ref[0, 0, i, 0]
        cy = cen_ref[0, 1, i, 0]
        cz = cen_ref[0, 2, i, 0]
        dx = x - cx
        dy = y - cy
        dz = z - cz
        d2 = dx * dx + dy * dy + dz * dz
        cpad = (cx * (cio == 0) + cy * (cio == 1) + cz * (cio == 2)
                ).astype(jnp.float32)

        for (r2, ns), out_ref in zip(branches, out_refs):
            rowsel_t, grank, mask, k_col, total = _selection(
                d2, r2, ns, tl, tr, rows)
            big = jnp.concatenate([grank, mask.astype(jnp.float32)], axis=1)
            gath = jnp.dot(rowsel_t, big,
                           preferred_element_type=jnp.float32)
            hot = jnp.logical_and(gath[:, 0:128] == k_col,
                                  gath[:, 128:256] > 0.5).astype(jnp.float32)
            l_col = jnp.dot(hot, lane_col,
                            preferred_element_type=jnp.float32)
            r_col = jnp.dot(rowsel_t, row_col,
                            preferred_element_type=jnp.float32)
            idx_col = r_col * 128.0 + l_col
            idx_col = jnp.where(k_col < total, idx_col, idx_col[0, 0])
            jio = jax.lax.broadcasted_iota(jnp.int32, (ns, n), 1).astype(jnp.float32)
            ph = (jio == idx_col).astype(jnp.float32)
            g = jnp.dot(ph, tab, precision=jax.lax.Precision.HIGHEST,
                        preferred_element_type=jnp.float32) - cpad
            out_ref[0, i] = g


def _ball_group2(xyz, new_xyz, table, radii, nsamples):
    b, n, _ = xyz.shape
    p = new_xyz.shape[1]
    c = table.shape[-1]
    rows = n // 128
    pc = 8
    branches = tuple((np.float32(r * r), ns) for r, ns in zip(radii, nsamples))
    xyz_r = jnp.transpose(xyz, (0, 2, 1)).reshape(b, 3, rows, 128)
    cen_r = jnp.transpose(new_xyz, (0, 2, 1)).reshape(b, 3, p, 1)
    tl = jnp.asarray(np.triu(np.ones((128, 128), np.float32), 1), dtype=jnp.bfloat16)
    tr = jnp.asarray(np.tril(np.ones((rows, rows), np.float32), -1))
    return pl.pallas_call(
        functools.partial(_bg2_kernel, branches=branches, pc=pc, rows=rows,
                          c=c),
        grid=(b, p // pc),
        in_specs=[
            pl.BlockSpec((1, 3, rows, 128), lambda i, j: (i, 0, 0, 0)),
            pl.BlockSpec((1, 3, pc, 1), lambda i, j: (i, 0, j, 0)),
            pl.BlockSpec((128, 128), lambda i, j: (0, 0)),
            pl.BlockSpec((rows, rows), lambda i, j: (0, 0)),
            pl.BlockSpec((1, n, c), lambda i, j: (i, 0, 0)),
        ],
        out_specs=[
            pl.BlockSpec((1, pc, ns, c), lambda i, j: (i, j, 0, 0))
            for _, ns in branches
        ],
        out_shape=[
            jax.ShapeDtypeStruct((b, p, ns, c), jnp.float32)
            for _, ns in branches
        ],
    )(xyz_r, cen_r, tl, tr, table)


def _gather(points, idx):
    return jax.vmap(lambda pts, i: pts[i])(points, idx)


def _mlp_max_kernel(g_ref, w1_ref, b1_ref, w2_ref, b2_ref, w3_ref, b3_ref,
                    out_ref, *, pc, s):
    cin = g_ref.shape[-1]
    x = g_ref[0].reshape(pc * s, cin)
    x = jnp.maximum(
        jnp.dot(x, w1_ref[...], preferred_element_type=jnp.float32)
        + b1_ref[0], 0.0)
    x = jnp.maximum(
        jnp.dot(x, w2_ref[...], preferred_element_type=jnp.float32)
        + b2_ref[0], 0.0)
    x = jnp.maximum(
        jnp.dot(x, w3_ref[...], preferred_element_type=jnp.float32)
        + b3_ref[0], 0.0)
    h3 = x.shape[-1]
    out_ref[0] = x.reshape(pc, s, h3).max(axis=1)


def _mlp_max(g, params):
    b, p, s, cin = g.shape
    (w1, b1), (w2, b2), (w3, b3) = params
    h3 = w3.shape[1]
    pc = max(1, min(p, 2048 // s))
    while p % pc:
        pc -= 1
    grid = (b, p // pc)
    b1r, b2r, b3r = (v.reshape(1, -1) for v in (b1, b2, b3))

    def wspec(w):
        return pl.BlockSpec(w.shape, lambda i, j: (0,) * w.ndim)

    return pl.pallas_call(
        functools.partial(_mlp_max_kernel, pc=pc, s=s),
        grid=grid,
        in_specs=[
            pl.BlockSpec((1, pc, s, cin), lambda i, j: (i, j, 0, 0)),
            wspec(w1), wspec(b1r), wspec(w2), wspec(b2r), wspec(w3), wspec(b3r),
        ],
        out_specs=pl.BlockSpec((1, pc, h3), lambda i, j: (i, j, 0)),
        out_shape=jax.ShapeDtypeStruct((b, p, h3), jnp.float32),
    )(g, w1, b1r, w2, b2r, w3, b3r)


def kernel(xyz, sa1_params, sa2_params, sa3_params):
    idx1 = _fps(xyz, 512)
    l1_xyz = _gather(xyz, idx1)
    g1 = _ball_group1(xyz, l1_xyz, [0.1, 0.2, 0.4], [32, 64, 128])
    l1_f = jnp.concatenate(
        [_mlp_max(g, p) for g, p in zip(g1, sa1_params)], axis=-1)

    idx2 = _fps(l1_xyz, 128)
    l2_xyz = _gather(l1_xyz, idx2)
    table = jnp.concatenate([l1_xyz, l1_f], axis=-1)
    g2 = _ball_group2(l1_xyz, l2_xyz, table, [0.4, 0.8], [64, 128])
    l2_f = jnp.concatenate(
        [_mlp_max(g, p) for g, p in zip(g2, sa2_params)], axis=-1)

    g3 = jnp.concatenate([l2_xyz, l2_f], axis=-1)[:, None, :, :]
    l3 = _mlp_max(g3, sa3_params)
    return jnp.transpose(l3, (0, 2, 1))


# speedup vs baseline: 6.7701x; 1.0051x over previous
import functools

import numpy as np
import jax
import jax.numpy as jnp
from jax.experimental import pallas as pl


def _fps_kernel(xyz_ref, out_ref, *, npoint, rows, cols):
    x = xyz_ref[0, 0]
    y = xyz_ref[0, 1]
    z = xyz_ref[0, 2]
    ri = jax.lax.broadcasted_iota(jnp.int32, (rows, cols), 0)
    ci = jax.lax.broadcasted_iota(jnp.int32, (rows, cols), 1)
    jf = ri * cols + ci

    def step(i, carry):
        dists, far = carry
        out_ref[0, pl.ds(i, 1), :] = jnp.full((1, 1), far, jnp.int32)
        onehot = (jf == far).astype(jnp.float32)
        cx = jnp.sum(x * onehot)
        cy = jnp.sum(y * onehot)
        cz = jnp.sum(z * onehot)
        dx = x - cx
        dy = y - cy
        dz = z - cz
        d = dx * dx + dy * dy + dz * dz
        dists = jnp.minimum(dists, d)
        mx = jnp.max(dists)
        cand = jnp.where(dists == mx, jf, jnp.int32(2 ** 30))
        return dists, jnp.min(cand)

    init = (jnp.full((rows, cols), 1e10, jnp.float32), jnp.int32(0))
    jax.lax.fori_loop(0, npoint, step, init)


def _fps(xyz, npoint):
    b, n, _ = xyz.shape
    rows = 8
    cols = n // rows
    xyz_r = jnp.transpose(xyz, (0, 2, 1)).reshape(b, 3, rows, cols)
    out = pl.pallas_call(
        functools.partial(_fps_kernel, npoint=npoint, rows=rows, cols=cols),
        grid=(b,),
        in_specs=[pl.BlockSpec((1, 3, rows, cols), lambda i: (i, 0, 0, 0))],
        out_specs=pl.BlockSpec((1, npoint, 1), lambda i: (i, 0, 0)),
        out_shape=jax.ShapeDtypeStruct((b, npoint, 1), jnp.int32),
    )(xyz_r)
    return out[:, :, 0]


def _selection(d2, r2, ns, tl, tr, rows):
    mask = (d2 < r2).astype(jnp.bfloat16)
    cs = jnp.dot(mask, tl, preferred_element_type=jnp.float32)
    rc = cs[:, 127:128] + mask[:, 127:128].astype(jnp.float32)
    row_off = jnp.dot(tr.astype(jnp.float32), rc,
                      preferred_element_type=jnp.float32)
    total = row_off[rows - 1, 0] + rc[rows - 1, 0]
    grank = cs + row_off
    kio = jax.lax.broadcasted_iota(jnp.int32, (rows, ns), 1).astype(jnp.float32)
    c1 = (row_off <= kio).astype(jnp.float32)
    c1s = jnp.concatenate([c1[1:], jnp.zeros((1, ns), jnp.float32)], axis=0)
    rowsel_t = (c1 - c1s).T
    k_col = jax.lax.broadcasted_iota(jnp.int32, (ns, 1), 0).astype(jnp.float32)
    return rowsel_t, grank, mask, k_col, total


def _bg1_kernel(xyz_ref, cen_ref, tl_ref, tr_ref, *out_refs, branches, pc,
                rows):
    x = xyz_ref[0, 0]
    y = xyz_ref[0, 1]
    z = xyz_ref[0, 2]
    tl = tl_ref[...]
    tr = tr_ref[...]
    xyz_cat = jnp.concatenate([x, y, z], axis=1)

    for i in range(pc):
        cx = cen_ref[0, 0, i, 0]
        cy = cen_ref[0, 1, i, 0]
        cz = cen_ref[0, 2, i, 0]
        dx = x - cx
        dy = y - cy
        dz = z - cz
        d2 = dx * dx + dy * dy + dz * dz

        for (r2, ns), out_ref in zip(branches, out_refs):
            rowsel_t, grank, mask, k_col, total = _selection(
                d2, r2, ns, tl, tr, rows)
            big = jnp.concatenate([grank, mask.astype(jnp.float32)], axis=1)
            gath = jnp.dot(rowsel_t, big,
                           preferred_element_type=jnp.float32)
            gxyz = jnp.dot(rowsel_t, xyz_cat,
                           precision=jax.lax.Precision.HIGHEST,
                           preferred_element_type=jnp.float32)
            q = gath[:, 0:128]
            qm = gath[:, 128:256]
            hot = jnp.logical_and(q == k_col, qm > 0.5).astype(jnp.float32)
            gx = jnp.sum(gxyz[:, 0:128] * hot, axis=1, keepdims=True)
            gy = jnp.sum(gxyz[:, 128:256] * hot, axis=1, keepdims=True)
            gz = jnp.sum(gxyz[:, 256:384] * hot, axis=1, keepdims=True)
            valid = k_col < total
            gx = jnp.where(valid, gx, gx[0, 0])
            gy = jnp.where(valid, gy, gy[0, 0])
            gz = jnp.where(valid, gz, gz[0, 0])
            g = jnp.concatenate(
                [gx - cx, gy - cy, gz - cz, gx, gy, gz], axis=1)
            out_ref[0, i] = g


def _ball_group1(xyz, new_xyz, radii, nsamples):
    b, n, _ = xyz.shape
    p = new_xyz.shape[1]
    rows = n // 128
    pc = 16
    branches = tuple((np.float32(r * r), ns) for r, ns in zip(radii, nsamples))
    xyz_r = jnp.transpose(xyz, (0, 2, 1)).reshape(b, 3, rows, 128)
    cen_r = jnp.transpose(new_xyz, (0, 2, 1)).reshape(b, 3, p, 1)
    tl = jnp.asarray(np.triu(np.ones((128, 128), np.float32), 1), dtype=jnp.bfloat16)
    tr = jnp.asarray(np.tril(np.ones((rows, rows), np.float32), -1))
    return pl.pallas_call(
        functools.partial(_bg1_kernel, branches=branches, pc=pc, rows=rows),
        grid=(b, p // pc),
        in_specs=[
            pl.BlockSpec((1, 3, rows, 128), lambda i, j: (i, 0, 0, 0)),
            pl.BlockSpec((1, 3, pc, 1), lambda i, j: (i, 0, j, 0)),
            pl.BlockSpec((128, 128), lambda i, j: (0, 0)),
            pl.BlockSpec((rows, rows), lambda i, j: (0, 0)),
        ],
        out_specs=[
            pl.BlockSpec((1, pc, ns, 6), lambda i, j: (i, j, 0, 0))
            for _, ns in branches
        ],
        out_shape=[
            jax.ShapeDtypeStruct((b, p, ns, 6), jnp.float32)
            for _, ns in branches
        ],
    )(xyz_r, cen_r, tl, tr)


def _bg2_kernel(xyz_ref, cen_ref, tl_ref, tr_ref, tab_ref, *out_refs,
                branches, pc, rows, c):
    x = xyz_ref[0, 0]
    y = xyz_ref[0, 1]
    z = xyz_ref[0, 2]
    tl = tl_ref[...]
    tr = tr_ref[...]
    tab = tab_ref[0]
    n = rows * 128
    lane_col = jax.lax.broadcasted_iota(jnp.int32, (128, 1), 0).astype(jnp.float32)
    row_col = jax.lax.broadcasted_iota(jnp.int32, (rows, 1), 0).astype(jnp.float32)
    cio = jax.lax.broadcasted_iota(jnp.int32, (1, c), 1)

    for i in range(pc):
        cx = cen_ref[0, 0, i, 0]
        cy = cen_ref[0, 1, i, 0]
        cz = cen_ref[0, 2, i, 0]
        dx = x - cx
        dy = y - cy
        dz = z - cz
        d2 = dx * dx + dy * dy + dz * dz
        cpad = (cx * (cio == 0) + cy * (cio == 1) + cz * (cio == 2)
                ).astype(jnp.float32)

        for (r2, ns), out_ref in zip(branches, out_refs):
            rowsel_t, grank, mask, k_col, total = _selection(
                d2, r2, ns, tl, tr, rows)
            big = jnp.concatenate([grank, mask.astype(jnp.float32)], axis=1)
            gath = jnp.dot(rowsel_t, big,
                           preferred_element_type=jnp.float32)
            hot = jnp.logical_and(gath[:, 0:128] == k_col,
                                  gath[:, 128:256] > 0.5).astype(jnp.float32)
            l_col = jnp.dot(hot, lane_col,
                            preferred_element_type=jnp.float32)
            r_col = jnp.dot(rowsel_t, row_col,
                            preferred_element_type=jnp.float32)
            idx_col = r_col * 128.0 + l_col
            idx_col = jnp.where(k_col < total, idx_col, idx_col[0, 0])
            jio = jax.lax.broadcasted_iota(jnp.int32, (ns, n), 1).astype(jnp.float32)
            ph = (jio == idx_col).astype(jnp.float32)
            g = jnp.dot(ph, tab, precision=jax.lax.Precision.HIGHEST,
                        preferred_element_type=jnp.float32) - cpad
            out_ref[0, i] = g


def _ball_group2(xyz, new_xyz, table, radii, nsamples):
    b, n, _ = xyz.shape
    p = new_xyz.shape[1]
    c = table.shape[-1]
    rows = n // 128
    pc = 16
    branches = tuple((np.float32(r * r), ns) for r, ns in zip(radii, nsamples))
    xyz_r = jnp.transpose(xyz, (0, 2, 1)).reshape(b, 3, rows, 128)
    cen_r = jnp.transpose(new_xyz, (0, 2, 1)).reshape(b, 3, p, 1)
    tl = jnp.asarray(np.triu(np.ones((128, 128), np.float32), 1), dtype=jnp.bfloat16)
    tr = jnp.asarray(np.tril(np.ones((rows, rows), np.float32), -1))
    return pl.pallas_call(
        functools.partial(_bg2_kernel, branches=branches, pc=pc, rows=rows,
                          c=c),
        grid=(b, p // pc),
        in_specs=[
            pl.BlockSpec((1, 3, rows, 128), lambda i, j: (i, 0, 0, 0)),
            pl.BlockSpec((1, 3, pc, 1), lambda i, j: (i, 0, j, 0)),
            pl.BlockSpec((128, 128), lambda i, j: (0, 0)),
            pl.BlockSpec((rows, rows), lambda i, j: (0, 0)),
            pl.BlockSpec((1, n, c), lambda i, j: (i, 0, 0)),
        ],
        out_specs=[
            pl.BlockSpec((1, pc, ns, c), lambda i, j: (i, j, 0, 0))
            for _, ns in branches
        ],
        out_shape=[
            jax.ShapeDtypeStruct((b, p, ns, c), jnp.float32)
            for _, ns in branches
        ],
    )(xyz_r, cen_r, tl, tr, table)


def _gather(points, idx):
    return jax.vmap(lambda pts, i: pts[i])(points, idx)


def _mlp_max_kernel(g_ref, w1_ref, b1_ref, w2_ref, b2_ref, w3_ref, b3_ref,
                    out_ref, *, pc, s):
    cin = g_ref.shape[-1]
    x = g_ref[0].reshape(pc * s, cin)
    x = jnp.maximum(
        jnp.dot(x, w1_ref[...], preferred_element_type=jnp.float32)
        + b1_ref[0], 0.0)
    x = jnp.maximum(
        jnp.dot(x, w2_ref[...], preferred_element_type=jnp.float32)
        + b2_ref[0], 0.0)
    x = jnp.maximum(
        jnp.dot(x, w3_ref[...], preferred_element_type=jnp.float32)
        + b3_ref[0], 0.0)
    h3 = x.shape[-1]
    out_ref[0] = x.reshape(pc, s, h3).max(axis=1)


def _mlp_max(g, params):
    b, p, s, cin = g.shape
    (w1, b1), (w2, b2), (w3, b3) = params
    h3 = w3.shape[1]
    pc = max(1, min(p, 2048 // s))
    while p % pc:
        pc -= 1
    grid = (b, p // pc)
    b1r, b2r, b3r = (v.reshape(1, -1) for v in (b1, b2, b3))

    def wspec(w):
        return pl.BlockSpec(w.shape, lambda i, j: (0,) * w.ndim)

    return pl.pallas_call(
        functools.partial(_mlp_max_kernel, pc=pc, s=s),
        grid=grid,
        in_specs=[
            pl.BlockSpec((1, pc, s, cin), lambda i, j: (i, j, 0, 0)),
            wspec(w1), wspec(b1r), wspec(w2), wspec(b2r), wspec(w3), wspec(b3r),
        ],
        out_specs=pl.BlockSpec((1, pc, h3), lambda i, j: (i, j, 0)),
        out_shape=jax.ShapeDtypeStruct((b, p, h3), jnp.float32),
    )(g, w1, b1r, w2, b2r, w3, b3r)


def kernel(xyz, sa1_params, sa2_params, sa3_params):
    idx1 = _fps(xyz, 512)
    l1_xyz = _gather(xyz, idx1)
    g1 = _ball_group1(xyz, l1_xyz, [0.1, 0.2, 0.4], [32, 64, 128])
    l1_f = jnp.concatenate(
        [_mlp_max(g, p) for g, p in zip(g1, sa1_params)], axis=-1)

    idx2 = _fps(l1_xyz, 128)
    l2_xyz = _gather(l1_xyz, idx2)
    table = jnp.concatenate([l1_xyz, l1_f], axis=-1)
    g2 = _ball_group2(l1_xyz, l2_xyz, table, [0.4, 0.8], [64, 128])
    l2_f = jnp.concatenate(
        [_mlp_max(g, p) for g, p in zip(g2, sa2_params)], axis=-1)

    g3 = jnp.concatenate([l2_xyz, l2_f], axis=-1)[:, None, :, :]
    l3 = _mlp_max(g3, sa3_params)
    return jnp.transpose(l3, (0, 2, 1))


# speedup vs baseline: 7.7490x; 1.1446x over previous
import functools

import numpy as np
import jax
import jax.numpy as jnp
from jax.experimental import pallas as pl


def _fps_kernel(xyz_ref, out_ref, *, npoint, b, rows, cols):
    x = xyz_ref[0]
    y = xyz_ref[1]
    z = xyz_ref[2]
    ri = jax.lax.broadcasted_iota(jnp.int32, (b, rows, cols), 1)
    ci = jax.lax.broadcasted_iota(jnp.int32, (b, rows, cols), 2)
    jf = ri * cols + ci

    def step(i, carry):
        dists, far = carry
        out_ref[:, pl.ds(i, 1), :] = far
        onehot = (jf == far).astype(jnp.float32)
        cx = jnp.sum(x * onehot, axis=(1, 2), keepdims=True)
        cy = jnp.sum(y * onehot, axis=(1, 2), keepdims=True)
        cz = jnp.sum(z * onehot, axis=(1, 2), keepdims=True)
        dx = x - cx
        dy = y - cy
        dz = z - cz
        d = dx * dx + dy * dy + dz * dz
        dists = jnp.minimum(dists, d)
        mx = jnp.max(dists, axis=(1, 2), keepdims=True)
        cand = jnp.where(dists == mx, jf, jnp.int32(2 ** 30))
        return dists, jnp.min(cand, axis=(1, 2), keepdims=True)

    init = (jnp.full((b, rows, cols), 1e10, jnp.float32),
            jnp.zeros((b, 1, 1), jnp.int32))
    jax.lax.fori_loop(0, npoint, step, init)


def _fps(xyz, npoint):
    b, n, _ = xyz.shape
    rows = 8
    cols = n // rows
    xyz_r = jnp.transpose(xyz, (0, 2, 1)).reshape(b, 3, rows, cols)
    xyz_r = jnp.transpose(xyz_r, (1, 0, 2, 3))
    out = pl.pallas_call(
        functools.partial(_fps_kernel, npoint=npoint, b=b, rows=rows,
                          cols=cols),
        in_specs=[pl.BlockSpec((3, b, rows, cols), lambda: (0, 0, 0, 0))],
        out_specs=pl.BlockSpec((b, npoint, 1), lambda: (0, 0, 0)),
        out_shape=jax.ShapeDtypeStruct((b, npoint, 1), jnp.int32),
    )(xyz_r)
    return out[:, :, 0]


def _selection(d2, r2, ns, tl, tr, rows):
    mask = (d2 < r2).astype(jnp.bfloat16)
    cs = jnp.dot(mask, tl, preferred_element_type=jnp.float32)
    rc = cs[:, 127:128] + mask[:, 127:128].astype(jnp.float32)
    row_off = jnp.dot(tr.astype(jnp.float32), rc,
                      preferred_element_type=jnp.float32)
    total = row_off[rows - 1, 0] + rc[rows - 1, 0]
    grank = cs + row_off
    kio = jax.lax.broadcasted_iota(jnp.int32, (rows, ns), 1).astype(jnp.float32)
    c1 = (row_off <= kio).astype(jnp.float32)
    c1s = jnp.concatenate([c1[1:], jnp.zeros((1, ns), jnp.float32)], axis=0)
    rowsel_t = (c1 - c1s).T
    k_col = jax.lax.broadcasted_iota(jnp.int32, (ns, 1), 0).astype(jnp.float32)
    return rowsel_t, grank, mask, k_col, total


def _bg1_kernel(xyz_ref, cen_ref, tl_ref, tr_ref, *out_refs, branches, pc,
                rows):
    x = xyz_ref[0, 0]
    y = xyz_ref[0, 1]
    z = xyz_ref[0, 2]
    tl = tl_ref[...]
    tr = tr_ref[...]
    xyz_cat = jnp.concatenate([x, y, z], axis=1)

    for i in range(pc):
        cx = cen_ref[0, 0, i, 0]
        cy = cen_ref[0, 1, i, 0]
        cz = cen_ref[0, 2, i, 0]
        dx = x - cx
        dy = y - cy
        dz = z - cz
        d2 = dx * dx + dy * dy + dz * dz

        for (r2, ns), out_ref in zip(branches, out_refs):
            rowsel_t, grank, mask, k_col, total = _selection(
                d2, r2, ns, tl, tr, rows)
            big = jnp.concatenate([grank, mask.astype(jnp.float32)], axis=1)
            gath = jnp.dot(rowsel_t, big,
                           preferred_element_type=jnp.float32)
            gxyz = jnp.dot(rowsel_t, xyz_cat,
                           precision=jax.lax.Precision.HIGHEST,
                           preferred_element_type=jnp.float32)
            q = gath[:, 0:128]
            qm = gath[:, 128:256]
            hot = jnp.logical_and(q == k_col, qm > 0.5).astype(jnp.float32)
            gx = jnp.sum(gxyz[:, 0:128] * hot, axis=1, keepdims=True)
            gy = jnp.sum(gxyz[:, 128:256] * hot, axis=1, keepdims=True)
            gz = jnp.sum(gxyz[:, 256:384] * hot, axis=1, keepdims=True)
            valid = k_col < total
            gx = jnp.where(valid, gx, gx[0, 0])
            gy = jnp.where(valid, gy, gy[0, 0])
            gz = jnp.where(valid, gz, gz[0, 0])
            g = jnp.concatenate(
                [gx - cx, gy - cy, gz - cz, gx, gy, gz], axis=1)
            out_ref[0, i] = g


def _ball_group1(xyz, new_xyz, radii, nsamples):
    b, n, _ = xyz.shape
    p = new_xyz.shape[1]
    rows = n // 128
    pc = 16
    branches = tuple((np.float32(r * r), ns) for r, ns in zip(radii, nsamples))
    xyz_r = jnp.transpose(xyz, (0, 2, 1)).reshape(b, 3, rows, 128)
    cen_r = jnp.transpose(new_xyz, (0, 2, 1)).reshape(b, 3, p, 1)
    tl = jnp.asarray(np.triu(np.ones((128, 128), np.float32), 1), dtype=jnp.bfloat16)
    tr = jnp.asarray(np.tril(np.ones((rows, rows), np.float32), -1))
    return pl.pallas_call(
        functools.partial(_bg1_kernel, branches=branches, pc=pc, rows=rows),
        grid=(b, p // pc),
        in_specs=[
            pl.BlockSpec((1, 3, rows, 128), lambda i, j: (i, 0, 0, 0)),
            pl.BlockSpec((1, 3, pc, 1), lambda i, j: (i, 0, j, 0)),
            pl.BlockSpec((128, 128), lambda i, j: (0, 0)),
            pl.BlockSpec((rows, rows), lambda i, j: (0, 0)),
        ],
        out_specs=[
            pl.BlockSpec((1, pc, ns, 6), lambda i, j: (i, j, 0, 0))
            for _, ns in branches
        ],
        out_shape=[
            jax.ShapeDtypeStruct((b, p, ns, 6), jnp.float32)
            for _, ns in branches
        ],
    )(xyz_r, cen_r, tl, tr)


def _bg2_kernel(xyz_ref, cen_ref, tl_ref, tr_ref, tab_ref, *out_refs,
                branches, pc, rows, c):
    x = xyz_ref[0, 0]
    y = xyz_ref[0, 1]
    z = xyz_ref[0, 2]
    tl = tl_ref[...]
    tr = tr_ref[...]
    tab = tab_ref[0]
    n = rows * 128
    lane_col = jax.lax.broadcasted_iota(jnp.int32, (128, 1), 0).astype(jnp.float32)
    row_col = jax.lax.broadcasted_iota(jnp.int32, (rows, 1), 0).astype(jnp.float32)
    cio = jax.lax.broadcasted_iota(jnp.int32, (1, c), 1)

    for i in range(pc):
        cx = cen_ref[0, 0, i, 0]
        cy = cen_ref[0, 1, i, 0]
        cz = cen_ref[0, 2, i, 0]
        dx = x - cx
        dy = y - cy
        dz = z - cz
        d2 = dx * dx + dy * dy + dz * dz
        cpad = (cx * (cio == 0) + cy * (cio == 1) + cz * (cio == 2)
                ).astype(jnp.float32)

        for (r2, ns), out_ref in zip(branches, out_refs):
            rowsel_t, grank, mask, k_col, total = _selection(
                d2, r2, ns, tl, tr, rows)
            big = jnp.concatenate([grank, mask.astype(jnp.float32)], axis=1)
            gath = jnp.dot(rowsel_t, big,
                           preferred_element_type=jnp.float32)
            hot = jnp.logical_and(gath[:, 0:128] == k_col,
                                  gath[:, 128:256] > 0.5).astype(jnp.float32)
            l_col = jnp.dot(hot, lane_col,
                            preferred_element_type=jnp.float32)
            r_col = jnp.dot(rowsel_t, row_col,
                            preferred_element_type=jnp.float32)
            idx_col = r_col * 128.0 + l_col
            idx_col = jnp.where(k_col < total, idx_col, idx_col[0, 0])
            jio = jax.lax.broadcasted_iota(jnp.int32, (ns, n), 1).astype(jnp.float32)
            ph = (jio == idx_col).astype(jnp.float32)
            g = jnp.dot(ph, tab, precision=jax.lax.Precision.HIGHEST,
                        preferred_element_type=jnp.float32) - cpad
            out_ref[0, i] = g


def _ball_group2(xyz, new_xyz, table, radii, nsamples):
    b, n, _ = xyz.shape
    p = new_xyz.shape[1]
    c = table.shape[-1]
    rows = n // 128
    pc = 16
    branches = tuple((np.float32(r * r), ns) for r, ns in zip(radii, nsamples))
    xyz_r = jnp.transpose(xyz, (0, 2, 1)).reshape(b, 3, rows, 128)
    cen_r = jnp.transpose(new_xyz, (0, 2, 1)).reshape(b, 3, p, 1)
    tl = jnp.asarray(np.triu(np.ones((128, 128), np.float32), 1), dtype=jnp.bfloat16)
    tr = jnp.asarray(np.tril(np.ones((rows, rows), np.float32), -1))
    return pl.pallas_call(
        functools.partial(_bg2_kernel, branches=branches, pc=pc, rows=rows,
                          c=c),
        grid=(b, p // pc),
        in_specs=[
            pl.BlockSpec((1, 3, rows, 128), lambda i, j: (i, 0, 0, 0)),
            pl.BlockSpec((1, 3, pc, 1), lambda i, j: (i, 0, j, 0)),
            pl.BlockSpec((128, 128), lambda i, j: (0, 0)),
            pl.BlockSpec((rows, rows), lambda i, j: (0, 0)),
            pl.BlockSpec((1, n, c), lambda i, j: (i, 0, 0)),
        ],
        out_specs=[
            pl.BlockSpec((1, pc, ns, c), lambda i, j: (i, j, 0, 0))
            for _, ns in branches
        ],
        out_shape=[
            jax.ShapeDtypeStruct((b, p, ns, c), jnp.float32)
            for _, ns in branches
        ],
    )(xyz_r, cen_r, tl, tr, table)


def _gather(points, idx):
    return jax.vmap(lambda pts, i: pts[i])(points, idx)


def _mlp_max_kernel(g_ref, w1_ref, b1_ref, w2_ref, b2_ref, w3_ref, b3_ref,
                    out_ref, *, pc, s):
    cin = g_ref.shape[-1]
    x = g_ref[0].reshape(pc * s, cin)
    x = jnp.maximum(
        jnp.dot(x, w1_ref[...], preferred_element_type=jnp.float32)
        + b1_ref[0], 0.0)
    x = jnp.maximum(
        jnp.dot(x, w2_ref[...], preferred_element_type=jnp.float32)
        + b2_ref[0], 0.0)
    x = jnp.maximum(
        jnp.dot(x, w3_ref[...], preferred_element_type=jnp.float32)
        + b3_ref[0], 0.0)
    h3 = x.shape[-1]
    out_ref[0] = x.reshape(pc, s, h3).max(axis=1)


def _mlp_max(g, params):
    b, p, s, cin = g.shape
    (w1, b1), (w2, b2), (w3, b3) = params
    h3 = w3.shape[1]
    pc = max(1, min(p, 2048 // s))
    while p % pc:
        pc -= 1
    grid = (b, p // pc)
    b1r, b2r, b3r = (v.reshape(1, -1) for v in (b1, b2, b3))

    def wspec(w):
        return pl.BlockSpec(w.shape, lambda i, j: (0,) * w.ndim)

    return pl.pallas_call(
        functools.partial(_mlp_max_kernel, pc=pc, s=s),
        grid=grid,
        in_specs=[
            pl.BlockSpec((1, pc, s, cin), lambda i, j: (i, j, 0, 0)),
            wspec(w1), wspec(b1r), wspec(w2), wspec(b2r), wspec(w3), wspec(b3r),
        ],
        out_specs=pl.BlockSpec((1, pc, h3), lambda i, j: (i, j, 0)),
        out_shape=jax.ShapeDtypeStruct((b, p, h3), jnp.float32),
    )(g, w1, b1r, w2, b2r, w3, b3r)


def kernel(xyz, sa1_params, sa2_params, sa3_params):
    idx1 = _fps(xyz, 512)
    l1_xyz = _gather(xyz, idx1)
    g1 = _ball_group1(xyz, l1_xyz, [0.1, 0.2, 0.4], [32, 64, 128])
    l1_f = jnp.concatenate(
        [_mlp_max(g, p) for g, p in zip(g1, sa1_params)], axis=-1)

    idx2 = _fps(l1_xyz, 128)
    l2_xyz = _gather(l1_xyz, idx2)
    table = jnp.concatenate([l1_xyz, l1_f], axis=-1)
    g2 = _ball_group2(l1_xyz, l2_xyz, table, [0.4, 0.8], [64, 128])
    l2_f = jnp.concatenate(
        [_mlp_max(g, p) for g, p in zip(g2, sa2_params)], axis=-1)

    g3 = jnp.concatenate([l2_xyz, l2_f], axis=-1)[:, None, :, :]
    l3 = _mlp_max(g3, sa3_params)
    return jnp.transpose(l3, (0, 2, 1))


# speedup vs baseline: 15.9916x; 2.0637x over previous
import functools

import numpy as np
import jax
import jax.numpy as jnp
from jax.experimental import pallas as pl


def _fps_kernel(xyz_ref, out_ref, *, npoint, b, rows, cols):
    x = xyz_ref[0]
    y = xyz_ref[1]
    z = xyz_ref[2]
    ri = jax.lax.broadcasted_iota(jnp.int32, (b, rows, cols), 1)
    ci = jax.lax.broadcasted_iota(jnp.int32, (b, rows, cols), 2)
    jf = ri * cols + ci

    def step(i, carry):
        dists, far = carry
        out_ref[:, pl.ds(i, 1), :] = far
        onehot = (jf == far).astype(jnp.float32)
        cx = jnp.sum(x * onehot, axis=(1, 2), keepdims=True)
        cy = jnp.sum(y * onehot, axis=(1, 2), keepdims=True)
        cz = jnp.sum(z * onehot, axis=(1, 2), keepdims=True)
        dx = x - cx
        dy = y - cy
        dz = z - cz
        d = dx * dx + dy * dy + dz * dz
        dists = jnp.minimum(dists, d)
        mx = jnp.max(dists, axis=(1, 2), keepdims=True)
        cand = jnp.where(dists == mx, jf, jnp.int32(2 ** 30))
        return dists, jnp.min(cand, axis=(1, 2), keepdims=True)

    init = (jnp.full((b, rows, cols), 1e10, jnp.float32),
            jnp.zeros((b, 1, 1), jnp.int32))
    jax.lax.fori_loop(0, npoint, step, init)


def _fps(xyz, npoint):
    b, n, _ = xyz.shape
    rows = 8
    cols = n // rows
    xyz_r = jnp.transpose(xyz, (0, 2, 1)).reshape(b, 3, rows, cols)
    xyz_r = jnp.transpose(xyz_r, (1, 0, 2, 3))
    out = pl.pallas_call(
        functools.partial(_fps_kernel, npoint=npoint, b=b, rows=rows,
                          cols=cols),
        in_specs=[pl.BlockSpec((3, b, rows, cols), lambda: (0, 0, 0, 0))],
        out_specs=pl.BlockSpec((b, npoint, 1), lambda: (0, 0, 0)),
        out_shape=jax.ShapeDtypeStruct((b, npoint, 1), jnp.int32),
    )(xyz_r)
    return out[:, :, 0]


def _prep(d2_3, r2, pc, rows, tl, tlr):
    mask3 = (d2_3 < r2).astype(jnp.float32)
    maskb = mask3.reshape(pc * rows, 128).astype(jnp.bfloat16)
    cs_all = jnp.dot(maskb, tl, preferred_element_type=jnp.float32)
    rc_row = jnp.sum(mask3, axis=2)
    row_off_row = jnp.dot(rc_row.astype(jnp.bfloat16), tlr,
                          preferred_element_type=jnp.float32)
    totals = row_off_row[:, rows - 1:rows] + rc_row[:, rows - 1:rows]
    return cs_all, mask3, row_off_row, totals


def _slot_sel(ro_row, k_col, ns):
    c1 = (ro_row <= k_col).astype(jnp.float32)
    c1s = jnp.concatenate(
        [c1[:, 1:], jnp.zeros((ns, 1), jnp.float32)], axis=1)
    rowsel_t = c1 - c1s
    ro_k = jnp.sum(rowsel_t * ro_row, axis=1, keepdims=True)
    return rowsel_t, ro_k


def _d2_block(xyz_ref, cen_ref, pc):
    x = xyz_ref[0, 0]
    y = xyz_ref[0, 1]
    z = xyz_ref[0, 2]
    cx = cen_ref[0, 0].reshape(pc, 1, 1)
    cy = cen_ref[0, 1].reshape(pc, 1, 1)
    cz = cen_ref[0, 2].reshape(pc, 1, 1)
    dx = x[None] - cx
    dy = y[None] - cy
    dz = z[None] - cz
    return x, y, z, (dx * dx + dy * dy + dz * dz)


def _bg1_kernel(xyz_ref, cen_ref, tl_ref, tlr_ref, *out_refs, branches, pc,
                rows):
    tl = tl_ref[...]
    tlr = tlr_ref[...]
    x, y, z, d2_3 = _d2_block(xyz_ref, cen_ref, pc)
    xyz_cat = jnp.concatenate([x, y, z], axis=1)
    k_cols = {ns: jax.lax.broadcasted_iota(jnp.int32, (ns, 1), 0)
              .astype(jnp.float32) for _, ns in branches}

    for (r2, ns), out_ref in zip(branches, out_refs):
        cs_all, mask3, row_off_row, totals = _prep(d2_3, r2, pc, rows, tl, tlr)
        k_col = k_cols[ns]
        for i in range(pc):
            rowsel_t, ro_k = _slot_sel(row_off_row[i:i + 1, :], k_col, ns)
            big = jnp.concatenate(
                [cs_all[i * rows:(i + 1) * rows], mask3[i]], axis=1)
            gath = jnp.dot(rowsel_t, big,
                           preferred_element_type=jnp.float32)
            gxyz = jnp.dot(rowsel_t, xyz_cat,
                           precision=jax.lax.Precision.HIGHEST,
                           preferred_element_type=jnp.float32)
            hot = jnp.logical_and(gath[:, 0:128] + ro_k == k_col,
                                  gath[:, 128:256] > 0.5).astype(jnp.float32)
            gx = jnp.sum(gxyz[:, 0:128] * hot, axis=1, keepdims=True)
            gy = jnp.sum(gxyz[:, 128:256] * hot, axis=1, keepdims=True)
            gz = jnp.sum(gxyz[:, 256:384] * hot, axis=1, keepdims=True)
            valid = k_col < totals[i:i + 1, 0:1]
            gx = jnp.where(valid, gx, gx[0, 0])
            gy = jnp.where(valid, gy, gy[0, 0])
            gz = jnp.where(valid, gz, gz[0, 0])
            cxv = cen_ref[0, 0, i, 0]
            cyv = cen_ref[0, 1, i, 0]
            czv = cen_ref[0, 2, i, 0]
            g = jnp.concatenate(
                [gx - cxv, gy - cyv, gz - czv, gx, gy, gz], axis=1)
            out_ref[0, i] = g


def _ball_group1(xyz, new_xyz, radii, nsamples):
    b, n, _ = xyz.shape
    p = new_xyz.shape[1]
    rows = n // 128
    pc = 16
    branches = tuple((np.float32(r * r), ns) for r, ns in zip(radii, nsamples))
    xyz_r = jnp.transpose(xyz, (0, 2, 1)).reshape(b, 3, rows, 128)
    cen_r = jnp.transpose(new_xyz, (0, 2, 1)).reshape(b, 3, p, 1)
    tl = jnp.asarray(np.triu(np.ones((128, 128), np.float32), 1),
                     dtype=jnp.bfloat16)
    tlr = jnp.asarray(np.triu(np.ones((rows, rows), np.float32), 1),
                      dtype=jnp.bfloat16)
    return pl.pallas_call(
        functools.partial(_bg1_kernel, branches=branches, pc=pc, rows=rows),
        grid=(b, p // pc),
        in_specs=[
            pl.BlockSpec((1, 3, rows, 128), lambda i, j: (i, 0, 0, 0)),
            pl.BlockSpec((1, 3, pc, 1), lambda i, j: (i, 0, j, 0)),
            pl.BlockSpec((128, 128), lambda i, j: (0, 0)),
            pl.BlockSpec((rows, rows), lambda i, j: (0, 0)),
        ],
        out_specs=[
            pl.BlockSpec((1, pc, ns, 6), lambda i, j: (i, j, 0, 0))
            for _, ns in branches
        ],
        out_shape=[
            jax.ShapeDtypeStruct((b, p, ns, 6), jnp.float32)
            for _, ns in branches
        ],
    )(xyz_r, cen_r, tl, tlr)


def _bg2_kernel(xyz_ref, cen_ref, tl_ref, tlr_ref, tab_ref, *out_refs,
                branches, pc, rows, c):
    tl = tl_ref[...]
    tlr = tlr_ref[...]
    tab = tab_ref[0]
    n = rows * 128
    x, y, z, d2_3 = _d2_block(xyz_ref, cen_ref, pc)
    laneio = jax.lax.broadcasted_iota(jnp.int32, (1, 128), 1).astype(jnp.float32)
    rowio = jax.lax.broadcasted_iota(jnp.int32, (1, rows), 1).astype(jnp.float32)
    cio = jax.lax.broadcasted_iota(jnp.int32, (1, c), 1)
    k_cols = {ns: jax.lax.broadcasted_iota(jnp.int32, (ns, 1), 0)
              .astype(jnp.float32) for _, ns in branches}

    for (r2, ns), out_ref in zip(branches, out_refs):
        cs_all, mask3, row_off_row, totals = _prep(d2_3, r2, pc, rows, tl, tlr)
        k_col = k_cols[ns]
        for i in range(pc):
            rowsel_t, ro_k = _slot_sel(row_off_row[i:i + 1, :], k_col, ns)
            big = jnp.concatenate(
                [cs_all[i * rows:(i + 1) * rows], mask3[i]], axis=1)
            gath = jnp.dot(rowsel_t, big,
                           preferred_element_type=jnp.float32)
            hot = jnp.logical_and(gath[:, 0:128] + ro_k == k_col,
                                  gath[:, 128:256] > 0.5).astype(jnp.float32)
            l_col = jnp.sum(hot * laneio, axis=1, keepdims=True)
            r_col = jnp.sum(rowsel_t * rowio, axis=1, keepdims=True)
            idx_col = r_col * 128.0 + l_col
            idx_col = jnp.where(k_col < totals[i:i + 1, 0:1], idx_col,
                                idx_col[0, 0])
            jio = jax.lax.broadcasted_iota(jnp.int32, (ns, n), 1).astype(jnp.float32)
            ph = (jio == idx_col).astype(jnp.float32)
            cxv = cen_ref[0, 0, i, 0]
            cyv = cen_ref[0, 1, i, 0]
            czv = cen_ref[0, 2, i, 0]
            cpad = (cxv * (cio == 0) + cyv * (cio == 1) + czv * (cio == 2)
                    ).astype(jnp.float32)
            g = jnp.dot(ph, tab, precision=jax.lax.Precision.HIGHEST,
                        preferred_element_type=jnp.float32) - cpad
            out_ref[0, i] = g


def _ball_group2(xyz, new_xyz, table, radii, nsamples):
    b, n, _ = xyz.shape
    p = new_xyz.shape[1]
    c = table.shape[-1]
    rows = n // 128
    pc = 16
    branches = tuple((np.float32(r * r), ns) for r, ns in zip(radii, nsamples))
    xyz_r = jnp.transpose(xyz, (0, 2, 1)).reshape(b, 3, rows, 128)
    cen_r = jnp.transpose(new_xyz, (0, 2, 1)).reshape(b, 3, p, 1)
    tl = jnp.asarray(np.triu(np.ones((128, 128), np.float32), 1),
                     dtype=jnp.bfloat16)
    tlr = jnp.asarray(np.triu(np.ones((rows, rows), np.float32), 1),
                      dtype=jnp.bfloat16)
    return pl.pallas_call(
        functools.partial(_bg2_kernel, branches=branches, pc=pc, rows=rows,
                          c=c),
        grid=(b, p // pc),
        in_specs=[
            pl.BlockSpec((1, 3, rows, 128), lambda i, j: (i, 0, 0, 0)),
            pl.BlockSpec((1, 3, pc, 1), lambda i, j: (i, 0, j, 0)),
            pl.BlockSpec((128, 128), lambda i, j: (0, 0)),
            pl.BlockSpec((rows, rows), lambda i, j: (0, 0)),
            pl.BlockSpec((1, n, c), lambda i, j: (i, 0, 0)),
        ],
        out_specs=[
            pl.BlockSpec((1, pc, ns, c), lambda i, j: (i, j, 0, 0))
            for _, ns in branches
        ],
        out_shape=[
            jax.ShapeDtypeStruct((b, p, ns, c), jnp.float32)
            for _, ns in branches
        ],
    )(xyz_r, cen_r, tl, tlr, table)


def _gather(points, idx):
    return jax.vmap(lambda pts, i: pts[i])(points, idx)


def _mlp_max_kernel(g_ref, w1_ref, b1_ref, w2_ref, b2_ref, w3_ref, b3_ref,
                    out_ref, *, pc, s):
    cin = g_ref.shape[-1]
    x = g_ref[0].reshape(pc * s, cin)
    x = jnp.maximum(
        jnp.dot(x, w1_ref[...], preferred_element_type=jnp.float32)
        + b1_ref[0], 0.0)
    x = jnp.maximum(
        jnp.dot(x, w2_ref[...], preferred_element_type=jnp.float32)
        + b2_ref[0], 0.0)
    x = jnp.maximum(
        jnp.dot(x, w3_ref[...], preferred_element_type=jnp.float32)
        + b3_ref[0], 0.0)
    h3 = x.shape[-1]
    out_ref[0] = x.reshape(pc, s, h3).max(axis=1)


def _mlp_max(g, params):
    b, p, s, cin = g.shape
    (w1, b1), (w2, b2), (w3, b3) = params
    h3 = w3.shape[1]
    pc = max(1, min(p, 2048 // s))
    while p % pc:
        pc -= 1
    grid = (b, p // pc)
    b1r, b2r, b3r = (v.reshape(1, -1) for v in (b1, b2, b3))

    def wspec(w):
        return pl.BlockSpec(w.shape, lambda i, j: (0,) * w.ndim)

    return pl.pallas_call(
        functools.partial(_mlp_max_kernel, pc=pc, s=s),
        grid=grid,
        in_specs=[
            pl.BlockSpec((1, pc, s, cin), lambda i, j: (i, j, 0, 0)),
            wspec(w1), wspec(b1r), wspec(w2), wspec(b2r), wspec(w3), wspec(b3r),
        ],
        out_specs=pl.BlockSpec((1, pc, h3), lambda i, j: (i, j, 0)),
        out_shape=jax.ShapeDtypeStruct((b, p, h3), jnp.float32),
    )(g, w1, b1r, w2, b2r, w3, b3r)


def kernel(xyz, sa1_params, sa2_params, sa3_params):
    idx1 = _fps(xyz, 512)
    l1_xyz = _gather(xyz, idx1)
    g1 = _ball_group1(xyz, l1_xyz, [0.1, 0.2, 0.4], [32, 64, 128])
    l1_f = jnp.concatenate(
        [_mlp_max(g, p) for g, p in zip(g1, sa1_params)], axis=-1)

    idx2 = _fps(l1_xyz, 128)
    l2_xyz = _gather(l1_xyz, idx2)
    table = jnp.concatenate([l1_xyz, l1_f], axis=-1)
    g2 = _ball_group2(l1_xyz, l2_xyz, table, [0.4, 0.8], [64, 128])
    l2_f = jnp.concatenate(
        [_mlp_max(g, p) for g, p in zip(g2, sa2_params)], axis=-1)

    g3 = jnp.concatenate([l2_xyz, l2_f], axis=-1)[:, None, :, :]
    l3 = _mlp_max(g3, sa3_params)
    return jnp.transpose(l3, (0, 2, 1))
